# baseline (speedup 1.0000x reference)
import functools

import jax
import jax.numpy as jnp
from jax import lax
from jax.experimental import pallas as pl
from jax.experimental.pallas import tpu as pltpu
from jax.experimental.pallas import tpu_sc as plsc

BS, CH, L, H, W = 2, 256, 8, 50, 50
R = 64
PH, PW = 7, 7
SCALE = 0.0625

D = CH * L
NPIX = BS * H * W
NBINS = R * PH * PW
NW = 32
BPW = NBINS // NW
K = 16
G = 4
LANES = 16
NCH = D // LANES


def _bin_geometry(rois):
    b = jnp.clip(jnp.round(rois[:, 0]).astype(jnp.int32), 0, BS - 1)
    rsw = jnp.round(rois[:, 1] * SCALE).astype(jnp.int32)
    rsh = jnp.round(rois[:, 2] * SCALE).astype(jnp.int32)
    rew = jnp.round(rois[:, 3] * SCALE).astype(jnp.int32)
    reh = jnp.round(rois[:, 4] * SCALE).astype(jnp.int32)
    roi_w = jnp.maximum(rew - rsw + 1, 1)
    roi_h = jnp.maximum(reh - rsh + 1, 1)
    p = jnp.arange(PH, dtype=jnp.int32)
    hs = jnp.clip(p[None] * roi_h[:, None] // PH + rsh[:, None], 0, H)
    he = jnp.clip(((p[None] + 1) * roi_h[:, None] + PH - 1) // PH + rsh[:, None], 0, H)
    ws = jnp.clip(p[None] * roi_w[:, None] // PW + rsw[:, None], 0, W)
    we = jnp.clip(((p[None] + 1) * roi_w[:, None] + PW - 1) // PW + rsw[:, None], 0, W)
    bh = (he[:, :, None] - hs[:, :, None]) * jnp.ones((1, 1, PW), jnp.int32)
    bw = (we[:, None, :] - ws[:, None, :]) * jnp.ones((1, PH, 1), jnp.int32)
    bh = bh.reshape(NBINS)
    bw = bw.reshape(NBINS)
    hs_b = jnp.broadcast_to(hs[:, :, None], (R, PH, PW)).reshape(NBINS)
    ws_b = jnp.broadcast_to(ws[:, None, :], (R, PH, PW)).reshape(NBINS)
    valid = (bh > 0) & (bw > 0)
    area = jnp.where(valid, bh * bw, 1)
    bw_c = jnp.maximum(bw, 1)
    bh_c = jnp.maximum(bh, 1)
    k = jnp.arange(K, dtype=jnp.int32)
    inv = jnp.take(jnp.array([65536, 32768, 21846, 16384], jnp.int32),
                   bw_c - 1)
    dh0 = (k[None, :] * inv[:, None]) >> 16
    dh = jnp.minimum(dh0, bh_c[:, None] - 1)
    dw = jnp.where(k[None, :] < area[:, None],
                   k[None, :] - dh0 * bw_c[:, None],
                   jnp.zeros((), jnp.int32))
    hh = jnp.clip(hs_b[:, None] + dh, 0, H - 1)
    ww = jnp.clip(ws_b[:, None] + dw, 0, W - 1)
    bb = jnp.broadcast_to(b[:, None, None], (R, PH * PW, K)).reshape(NBINS, K)
    idx = (bb * (H * W) + hh * W + ww).astype(jnp.int32)
    vmul = jnp.broadcast_to(
        valid.reshape(NBINS, 1).astype(jnp.float32), (NBINS, K))
    areab = jnp.broadcast_to(area[:, None], (NBINS, K))
    return idx, vmul, areab.astype(jnp.int32)


@functools.cache
def _make_sc_pool():
    mesh = plsc.VectorSubcoreMesh(core_axis_name="c", subcore_axis_name="s")

    @functools.partial(
        pl.kernel,
        out_type=jax.ShapeDtypeStruct((NBINS, 1, D), jnp.float32),
        mesh=mesh,
        compiler_params=pltpu.CompilerParams(
            needs_layout_passes=False, use_tc_tiling_on_sc=False),
        scratch_types=[
            pltpu.VMEM((BPW * K,), jnp.int32),
            pltpu.VMEM((BPW, K), jnp.float32),
            pltpu.VMEM((BPW, K), jnp.int32),
            pltpu.VMEM((BPW, K), jnp.int32),
            pltpu.VMEM((K, 1, D), jnp.float32),
            pltpu.VMEM((K, 1, D), jnp.float32),
            pltpu.VMEM((1, D), jnp.float32),
            pltpu.VMEM((1, D), jnp.float32),
            pltpu.SemaphoreType.DMA,
            pltpu.SemaphoreType.DMA,
            pltpu.SemaphoreType.DMA,
            pltpu.SemaphoreType.DMA,
        ],
    )
    def _sc_pool(table_hbm, idx_hbm, vmul_hbm, ngrp_hbm, pos_hbm, out_hbm,
                 idx_v, vmul_v, ngrp_v, pos_v, rows_a, rows_b, orow_a, orow_b,
                 sem_a, sem_b, osem_a, osem_b):
        wid = lax.axis_index("s") * 2 + lax.axis_index("c")
        base = wid * BPW
        pltpu.sync_copy(idx_hbm.at[wid, 0], idx_v)
        pltpu.sync_copy(vmul_hbm.at[wid], vmul_v)
        pltpu.sync_copy(ngrp_hbm.at[wid], ngrp_v)
        pltpu.sync_copy(pos_hbm.at[wid], pos_v)

        def area_of(i):
            return jnp.max(ngrp_v[i, :])

        def issue(i, buf, sem):
            ar = area_of(i)
            off = pl.multiple_of(i * K, 8)
            for kk in range(1, K + 1):

                @pl.when(ar == kk)
                def _v(kk=kk):
                    pltpu.async_copy(
                        table_hbm.at[idx_v.at[pl.ds(off, kk)]],
                        buf.at[pl.ds(0, kk)], sem)

        def drain(i, buf, sem):
            ar = area_of(i)
            off = pl.multiple_of(i * K, 8)
            for kk in range(1, K + 1):

                @pl.when(ar == kk)
                def _v(kk=kk):
                    pltpu.make_async_copy(
                        table_hbm.at[idx_v.at[pl.ds(off, kk)]],
                        buf.at[pl.ds(0, kk)], sem).wait()

        def compute(i, buf, orow, osem, first):
            ar = area_of(i)
            fvec = vmul_v[i, :]

            @pl.when(jnp.logical_not(first))
            def _dr():
                pltpu.make_async_copy(out_hbm.at[base], orow, osem).wait()

            for kk in range(1, K + 1):

                @pl.when(ar == kk)
                def _variant(kk=kk):
                    @pl.loop(0, NCH)
                    def _d(d):
                        sl = pl.ds(d * LANES, LANES)
                        vals = [buf[r, 0, sl] for r in range(kk)]
                        while len(vals) > 1:
                            vals = [jnp.maximum(vals[a], vals[a + 1])
                                    for a in range(0, len(vals) - 1, 2)] + (
                                        [vals[-1]] if len(vals) % 2 else [])
                        orow[0, sl] = vals[0] * fvec

            pltpu.async_copy(orow, out_hbm.at[jnp.max(pos_v[i, :])], osem)

        issue(0, rows_a, sem_a)

        @pl.loop(0, BPW, step=2)
        def _bin_loop(i):
            issue(i + 1, rows_b, sem_b)
            drain(i, rows_a, sem_a)
            compute(i, rows_a, orow_a, osem_a, i == 0)

            @pl.when(i + 2 < BPW)
            def _pf():
                issue(i + 2, rows_a, sem_a)

            drain(i + 1, rows_b, sem_b)
            compute(i + 1, rows_b, orow_b, osem_b, i == 0)

        pltpu.make_async_copy(out_hbm.at[base], orow_a, osem_a).wait()
        pltpu.make_async_copy(out_hbm.at[base], orow_b, osem_b).wait()

    return _sc_pool


def kernel(input, rois):
    table = (jnp.transpose(input, (0, 3, 4, 2, 1))
             .reshape(BS, H, W, L, CH // 128, 128)
             .transpose(0, 1, 2, 4, 3, 5)
             .reshape(NPIX, D))
    idx, vmul, ngrp = _bin_geometry(rois)
    order = jnp.argsort(-ngrp[:, 0]).astype(jnp.int32)
    flatp = order.reshape(BPW, NW).T.reshape(NBINS)
    posb = jnp.broadcast_to(flatp[:, None], (NBINS, K))
    out = _make_sc_pool()(table.reshape(NPIX, 1, D),
                          idx[flatp].reshape(NW, 1, BPW * K),
                          vmul[flatp].reshape(NW, BPW, K),
                          ngrp[flatp].reshape(NW, BPW, K),
                          posb.reshape(NW, BPW, K))
    out = out.reshape(R, PH, PW, CH // 128, L, 128)
    return jnp.transpose(out, (0, 3, 5, 4, 1, 2)).reshape(R, CH, L, PH, PW)


# speedup vs baseline: 1.0115x; 1.0115x over previous
import functools

import jax
import jax.numpy as jnp
from jax import lax
from jax.experimental import pallas as pl
from jax.experimental.pallas import tpu as pltpu
from jax.experimental.pallas import tpu_sc as plsc

BS, CH, L, H, W = 2, 256, 8, 50, 50
R = 64
PH, PW = 7, 7
SCALE = 0.0625

D = CH * L
NPIX = BS * H * W
NBINS = R * PH * PW
NW = 32
BPW = NBINS // NW
K = 16
G = 4
LANES = 16
NCH = D // LANES


def _bin_geometry(rois):
    b = jnp.clip(jnp.round(rois[:, 0]).astype(jnp.int32), 0, BS - 1)
    rsw = jnp.round(rois[:, 1] * SCALE).astype(jnp.int32)
    rsh = jnp.round(rois[:, 2] * SCALE).astype(jnp.int32)
    rew = jnp.round(rois[:, 3] * SCALE).astype(jnp.int32)
    reh = jnp.round(rois[:, 4] * SCALE).astype(jnp.int32)
    roi_w = jnp.maximum(rew - rsw + 1, 1)
    roi_h = jnp.maximum(reh - rsh + 1, 1)
    p = jnp.arange(PH, dtype=jnp.int32)
    hs = jnp.clip(p[None] * roi_h[:, None] // PH + rsh[:, None], 0, H)
    he = jnp.clip(((p[None] + 1) * roi_h[:, None] + PH - 1) // PH + rsh[:, None], 0, H)
    ws = jnp.clip(p[None] * roi_w[:, None] // PW + rsw[:, None], 0, W)
    we = jnp.clip(((p[None] + 1) * roi_w[:, None] + PW - 1) // PW + rsw[:, None], 0, W)
    bh = (he[:, :, None] - hs[:, :, None]) * jnp.ones((1, 1, PW), jnp.int32)
    bw = (we[:, None, :] - ws[:, None, :]) * jnp.ones((1, PH, 1), jnp.int32)
    bh = bh.reshape(NBINS)
    bw = bw.reshape(NBINS)
    hs_b = jnp.broadcast_to(hs[:, :, None], (R, PH, PW)).reshape(NBINS)
    ws_b = jnp.broadcast_to(ws[:, None, :], (R, PH, PW)).reshape(NBINS)
    valid = (bh > 0) & (bw > 0)
    area = jnp.where(valid, bh * bw, 1)
    bw_c = jnp.maximum(bw, 1)
    bh_c = jnp.maximum(bh, 1)
    k = jnp.arange(K, dtype=jnp.int32)
    inv = jnp.take(jnp.array([65536, 32768, 21846, 16384], jnp.int32),
                   bw_c - 1)
    dh0 = (k[None, :] * inv[:, None]) >> 16
    dh = jnp.minimum(dh0, bh_c[:, None] - 1)
    dw = jnp.where(k[None, :] < area[:, None],
                   k[None, :] - dh0 * bw_c[:, None],
                   jnp.zeros((), jnp.int32))
    hh = jnp.clip(hs_b[:, None] + dh, 0, H - 1)
    ww = jnp.clip(ws_b[:, None] + dw, 0, W - 1)
    bb = jnp.broadcast_to(b[:, None, None], (R, PH * PW, K)).reshape(NBINS, K)
    idx = (bb * (H * W) + hh * W + ww).astype(jnp.int32)
    vmul = jnp.broadcast_to(
        valid.reshape(NBINS, 1).astype(jnp.float32), (NBINS, K))
    areab = jnp.broadcast_to(area[:, None], (NBINS, K))
    return idx, vmul, areab.astype(jnp.int32)


@functools.cache
def _make_sc_pool():
    mesh = plsc.VectorSubcoreMesh(core_axis_name="c", subcore_axis_name="s")

    @functools.partial(
        pl.kernel,
        out_type=jax.ShapeDtypeStruct((NBINS, 1, D), jnp.float32),
        mesh=mesh,
        compiler_params=pltpu.CompilerParams(
            needs_layout_passes=False, use_tc_tiling_on_sc=False),
        scratch_types=[
            pltpu.VMEM((BPW * K,), jnp.int32),
            pltpu.VMEM((BPW, K), jnp.float32),
            pltpu.VMEM((BPW, K), jnp.int32),
            pltpu.VMEM((K, 1, D), jnp.float32),
            pltpu.VMEM((K, 1, D), jnp.float32),
            pltpu.VMEM((1, D), jnp.float32),
            pltpu.VMEM((1, D), jnp.float32),
            pltpu.SemaphoreType.DMA,
            pltpu.SemaphoreType.DMA,
            pltpu.SemaphoreType.DMA,
            pltpu.SemaphoreType.DMA,
        ],
    )
    def _sc_pool(table_hbm, idx_hbm, vmul_hbm, ngrp_hbm, out_hbm,
                 idx_v, vmul_v, ngrp_v, rows_a, rows_b, orow_a, orow_b,
                 sem_a, sem_b, osem_a, osem_b):
        wid = lax.axis_index("s") * 2 + lax.axis_index("c")
        base = wid * BPW
        pltpu.sync_copy(idx_hbm.at[wid, 0], idx_v)
        pltpu.sync_copy(vmul_hbm.at[wid], vmul_v)
        pltpu.sync_copy(ngrp_hbm.at[wid], ngrp_v)

        def area_of(i):
            return jnp.max(ngrp_v[i, :])

        def issue(i, buf, sem):
            ar = area_of(i)
            off = pl.multiple_of(i * K, 8)
            for kk in range(1, K + 1):

                @pl.when(ar == kk)
                def _v(kk=kk):
                    pltpu.async_copy(
                        table_hbm.at[idx_v.at[pl.ds(off, kk)]],
                        buf.at[pl.ds(0, kk)], sem)

        def drain(i, buf, sem):
            ar = area_of(i)
            off = pl.multiple_of(i * K, 8)
            for kk in range(1, K + 1):

                @pl.when(ar == kk)
                def _v(kk=kk):
                    pltpu.make_async_copy(
                        table_hbm.at[idx_v.at[pl.ds(off, kk)]],
                        buf.at[pl.ds(0, kk)], sem).wait()

        def compute(i, buf, orow, osem, first):
            ar = area_of(i)
            fvec = vmul_v[i, :]

            @pl.when(jnp.logical_not(first))
            def _dr():
                pltpu.make_async_copy(out_hbm.at[base], orow, osem).wait()

            for kk in range(1, K + 1):

                @pl.when(ar == kk)
                def _variant(kk=kk):
                    @pl.loop(0, NCH)
                    def _d(d):
                        sl = pl.ds(d * LANES, LANES)
                        vals = [buf[r, 0, sl] for r in range(kk)]
                        while len(vals) > 1:
                            vals = [jnp.maximum(vals[a], vals[a + 1])
                                    for a in range(0, len(vals) - 1, 2)] + (
                                        [vals[-1]] if len(vals) % 2 else [])
                        orow[0, sl] = vals[0] * fvec

            pltpu.async_copy(orow, out_hbm.at[base + i], osem)

        issue(0, rows_a, sem_a)

        @pl.loop(0, BPW, step=2)
        def _bin_loop(i):
            issue(i + 1, rows_b, sem_b)
            drain(i, rows_a, sem_a)
            compute(i, rows_a, orow_a, osem_a, i == 0)

            @pl.when(i + 2 < BPW)
            def _pf():
                issue(i + 2, rows_a, sem_a)

            drain(i + 1, rows_b, sem_b)
            compute(i + 1, rows_b, orow_b, osem_b, i == 0)

        pltpu.make_async_copy(out_hbm.at[base], orow_a, osem_a).wait()
        pltpu.make_async_copy(out_hbm.at[base], orow_b, osem_b).wait()

    return _sc_pool


def kernel(input, rois):
    table = (jnp.transpose(input, (0, 3, 4, 2, 1))
             .reshape(BS, H, W, L, CH // 128, 128)
             .transpose(0, 1, 2, 4, 3, 5)
             .reshape(NPIX, D))
    idx, vmul, ngrp = _bin_geometry(rois)
    out = _make_sc_pool()(table.reshape(NPIX, 1, D), idx.reshape(NW, 1, BPW * K),
                          vmul.reshape(NW, BPW, K), ngrp.reshape(NW, BPW, K))
    out = out.reshape(R, PH, PW, CH // 128, L, 128)
    return jnp.transpose(out, (0, 3, 5, 4, 1, 2)).reshape(R, CH, L, PH, PW)


# speedup vs baseline: 1.0128x; 1.0013x over previous
import functools

import jax
import jax.numpy as jnp
from jax import lax
from jax.experimental import pallas as pl
from jax.experimental.pallas import tpu as pltpu
from jax.experimental.pallas import tpu_sc as plsc

BS, CH, L, H, W = 2, 256, 8, 50, 50
R = 64
PH, PW = 7, 7
SCALE = 0.0625

D = CH * L
NPIX = BS * H * W
NBINS = R * PH * PW
NW = 32
BPW = NBINS // NW
K = 16
LANES = 16
NCH = D // LANES


def _bin_geometry(rois):
    b = jnp.clip(jnp.round(rois[:, 0]).astype(jnp.int32), 0, BS - 1)
    rsw = jnp.round(rois[:, 1] * SCALE).astype(jnp.int32)
    rsh = jnp.round(rois[:, 2] * SCALE).astype(jnp.int32)
    rew = jnp.round(rois[:, 3] * SCALE).astype(jnp.int32)
    reh = jnp.round(rois[:, 4] * SCALE).astype(jnp.int32)
    roi_w = jnp.maximum(rew - rsw + 1, 1)
    roi_h = jnp.maximum(reh - rsh + 1, 1)
    p = jnp.arange(PH, dtype=jnp.int32)
    hs = jnp.clip(p[None] * roi_h[:, None] // PH + rsh[:, None], 0, H)
    he = jnp.clip(((p[None] + 1) * roi_h[:, None] + PH - 1) // PH + rsh[:, None], 0, H)
    ws = jnp.clip(p[None] * roi_w[:, None] // PW + rsw[:, None], 0, W)
    we = jnp.clip(((p[None] + 1) * roi_w[:, None] + PW - 1) // PW + rsw[:, None], 0, W)
    bh = (he[:, :, None] - hs[:, :, None]) * jnp.ones((1, 1, PW), jnp.int32)
    bw = (we[:, None, :] - ws[:, None, :]) * jnp.ones((1, PH, 1), jnp.int32)
    bh = bh.reshape(NBINS)
    bw = bw.reshape(NBINS)
    hs_b = jnp.broadcast_to(hs[:, :, None], (R, PH, PW)).reshape(NBINS)
    ws_b = jnp.broadcast_to(ws[:, None, :], (R, PH, PW)).reshape(NBINS)
    valid = (bh > 0) & (bw > 0)
    area = jnp.where(valid, bh * bw, 1)
    bw_c = jnp.maximum(bw, 1)
    bh_c = jnp.maximum(bh, 1)
    k = jnp.arange(K, dtype=jnp.int32)
    inv = jnp.take(jnp.array([65536, 32768, 21846, 16384], jnp.int32),
                   bw_c - 1)
    dh0 = (k[None, :] * inv[:, None]) >> 16
    dh = jnp.minimum(dh0, bh_c[:, None] - 1)
    dw = jnp.where(k[None, :] < area[:, None],
                   k[None, :] - dh0 * bw_c[:, None],
                   jnp.zeros((), jnp.int32))
    hh = jnp.clip(hs_b[:, None] + dh, 0, H - 1)
    ww = jnp.clip(ws_b[:, None] + dw, 0, W - 1)
    bb = jnp.broadcast_to(b[:, None, None], (R, PH * PW, K)).reshape(NBINS, K)
    idx = (bb * (H * W) + hh * W + ww).astype(jnp.int32)
    vmul = jnp.broadcast_to(
        valid.reshape(NBINS, 1).astype(jnp.float32), (NBINS, K))
    areab = jnp.broadcast_to(area[:, None], (NBINS, K))
    return idx, vmul, areab.astype(jnp.int32)


@functools.cache
def _make_sc_pool():
    mesh = plsc.VectorSubcoreMesh(core_axis_name="c", subcore_axis_name="s")

    @functools.partial(
        pl.kernel,
        out_type=jax.ShapeDtypeStruct((NBINS, 1, D), jnp.float32),
        mesh=mesh,
        compiler_params=pltpu.CompilerParams(
            needs_layout_passes=False, use_tc_tiling_on_sc=False),
        scratch_types=[
            pltpu.VMEM((BPW * K,), jnp.int32),
            pltpu.VMEM((BPW, K), jnp.float32),
            pltpu.VMEM((BPW, K), jnp.int32),
            pltpu.VMEM((K, 1, D), jnp.float32),
            pltpu.VMEM((K, 1, D), jnp.float32),
            pltpu.VMEM((1, D), jnp.float32),
            pltpu.VMEM((1, D), jnp.float32),
            pltpu.SemaphoreType.DMA,
            pltpu.SemaphoreType.DMA,
            pltpu.SemaphoreType.DMA,
            pltpu.SemaphoreType.DMA,
        ],
    )
    def _sc_pool(table_hbm, idx_hbm, vmul_hbm, ngrp_hbm, out_hbm,
                 idx_v, vmul_v, ngrp_v, rows_a, rows_b, orow_a, orow_b,
                 sem_a, sem_b, osem_a, osem_b):
        wid = lax.axis_index("s") * 2 + lax.axis_index("c")
        base = wid * BPW
        pltpu.sync_copy(idx_hbm.at[wid, 0], idx_v)
        pltpu.sync_copy(vmul_hbm.at[wid], vmul_v)
        pltpu.sync_copy(ngrp_hbm.at[wid], ngrp_v)

        def area_of(i):
            return jnp.max(ngrp_v[i, :])

        def issue(i, buf, sem):
            ar = area_of(i)
            off = pl.multiple_of(i * K, 8)
            for kk in range(1, K + 1):

                @pl.when(ar == kk)
                def _v(kk=kk):
                    pltpu.async_copy(
                        table_hbm.at[idx_v.at[pl.ds(off, kk)]],
                        buf.at[pl.ds(0, kk)], sem)

        def drain(i, buf, sem):
            ar = area_of(i)
            off = pl.multiple_of(i * K, 8)
            for kk in range(1, K + 1):

                @pl.when(ar == kk)
                def _v(kk=kk):
                    pltpu.make_async_copy(
                        table_hbm.at[idx_v.at[pl.ds(off, kk)]],
                        buf.at[pl.ds(0, kk)], sem).wait()

        def compute(i, buf, orow, osem, first):
            ar = area_of(i)
            fvec = vmul_v[i, :]

            @pl.when(jnp.logical_not(first))
            def _dr():
                pltpu.make_async_copy(out_hbm.at[base], orow, osem).wait()

            for kk in range(1, K + 1):

                @pl.when(ar == kk)
                def _variant(kk=kk):
                    @pl.loop(0, NCH)
                    def _d(d):
                        sl = pl.ds(d * LANES, LANES)
                        vals = [buf[r, 0, sl] for r in range(kk)]
                        while len(vals) > 1:
                            vals = [jnp.maximum(vals[a], vals[a + 1])
                                    for a in range(0, len(vals) - 1, 2)] + (
                                        [vals[-1]] if len(vals) % 2 else [])
                        orow[0, sl] = vals[0] * fvec

            pltpu.async_copy(orow, out_hbm.at[base + i], osem)

        issue(0, rows_a, sem_a)

        @pl.loop(0, BPW, step=2)
        def _bin_loop(i):
            issue(i + 1, rows_b, sem_b)
            drain(i, rows_a, sem_a)
            compute(i, rows_a, orow_a, osem_a, i == 0)

            @pl.when(i + 2 < BPW)
            def _pf():
                issue(i + 2, rows_a, sem_a)

            drain(i + 1, rows_b, sem_b)
            compute(i + 1, rows_b, orow_b, osem_b, i == 0)

        pltpu.make_async_copy(out_hbm.at[base], orow_a, osem_a).wait()
        pltpu.make_async_copy(out_hbm.at[base], orow_b, osem_b).wait()

    return _sc_pool


def kernel(input, rois):
    table = (jnp.transpose(input, (0, 3, 4, 2, 1))
             .reshape(BS, H, W, L, CH // 128, 128)
             .transpose(0, 1, 2, 4, 3, 5)
             .reshape(NPIX, D))
    idx, vmul, ngrp = _bin_geometry(rois)
    out = _make_sc_pool()(table.reshape(NPIX, 1, D), idx.reshape(NW, 1, BPW * K),
                          vmul.reshape(NW, BPW, K), ngrp.reshape(NW, BPW, K))
    out = out.reshape(R, PH, PW, CH // 128, L, 128)
    return jnp.transpose(out, (0, 3, 5, 4, 1, 2)).reshape(R, CH, L, PH, PW)
